# baseline (speedup 1.0000x reference)
import functools

import jax
import jax.numpy as jnp
from jax import lax
from jax.experimental import pallas as pl
from jax.experimental.pallas import tpu as pltpu
from jax.experimental.pallas import tpu_sc as plsc

_NC = 2
_NS = 16
_NW = _NC * _NS
_C = 128
_K = 2
_L = 16


@functools.cache
def _make_gather(nidx: int, nblk: int):
    per_w = nidx // _NW
    nchunk = per_w // _C
    ngroups = nchunk // _K
    assert per_w % _C == 0 and nchunk % _K == 0 and ngroups % 2 == 0
    irows = per_w // 128
    orows_c = _C * 32 // 128
    mesh = plsc.VectorSubcoreMesh(core_axis_name="c", subcore_axis_name="s")

    @functools.partial(
        pl.kernel,
        mesh=mesh,
        out_type=jax.ShapeDtypeStruct((nidx * 32 // 128, 128), jnp.float32),
        scratch_types=[
            pltpu.VMEM((irows, 128), jnp.int32),
            pltpu.VMEM((irows, 128), jnp.int32),
            pltpu.VMEM((2, _K, _C, 128), jnp.float32),
            pltpu.VMEM((2, _K, orows_c, 128), jnp.float32),
            pltpu.SemaphoreType.DMA,
            pltpu.SemaphoreType.DMA,
            pltpu.SemaphoreType.DMA,
            pltpu.SemaphoreType.DMA,
        ],
        compiler_params=pltpu.CompilerParams(
            use_tc_tiling_on_sc=False, needs_layout_passes=False
        ),
    )
    def k(t2, x2, out2, idx_v, idxq_v, gath_v, out_v, gsa, gsb, ssa, ssb):
        wid = lax.axis_index("s") * _NC + lax.axis_index("c")
        pltpu.sync_copy(x2.at[pl.ds(wid * irows, irows)], idx_v)

        iota = lax.iota(jnp.int32, _L)

        def prep(r, carry):
            for g in range(128 // _L):
                v = idx_v[r, pl.ds(g * _L, _L)]
                idxq_v[r, pl.ds(g * _L, _L)] = lax.shift_right_logical(v, 2)
            return carry

        lax.fori_loop(0, irows, prep, 0)

        def gather_desc(grp, bufset, b, sem):
            c = grp * _K + b
            return pltpu.make_async_copy(
                t2.at[idxq_v.at[c]], gath_v.at[bufset, b], sem
            )

        def store_desc(grp, bufset, b, sem):
            c = grp * _K + b
            return pltpu.make_async_copy(
                out_v.at[bufset, b],
                out2.at[pl.ds((wid * nchunk + c) * orows_c, orows_c)],
                sem,
            )

        def fire(desc_fn, grp, bufset, sem):
            for b in range(_K):
                desc_fn(grp, bufset, b, sem).start()

        def drain(desc_fn, grp, bufset, sem):
            for b in range(_K):
                desc_fn(grp, bufset, b, sem).wait()

        orow0 = lax.shift_right_logical(iota, 2)
        ocol0 = (iota & 3) * 32

        def extract(grp, bufset):
            for b in range(_K):
                c = grp * _K + b

                def grp16(g, carry):
                    m = idx_v[c, pl.ds(g * _L, _L)] & 3
                    rowv = g * _L + iota
                    orowv = g * 4 + orow0
                    for w in range(32):
                        vals = plsc.load_gather(
                            gath_v.at[bufset, b], [rowv, m * 32 + w]
                        )
                        plsc.store_scatter(
                            out_v.at[bufset, b], [orowv, ocol0 + w], vals
                        )
                    return carry

                lax.fori_loop(0, _C // _L, grp16, 0)

        fire(gather_desc, 0, 0, gsa)
        fire(gather_desc, 1, 1, gsb)
        drain(gather_desc, 0, 0, gsa)
        extract(0, 0)
        fire(store_desc, 0, 0, ssa)

        def pair_body(p, carry):
            drain(store_desc, 2 * p, 0, ssa)
            fire(gather_desc, 2 * p + 2, 0, gsa)
            drain(gather_desc, 2 * p + 1, 1, gsb)
            extract(2 * p + 1, 1)
            fire(store_desc, 2 * p + 1, 1, ssb)
            drain(store_desc, 2 * p + 1, 1, ssb)
            fire(gather_desc, 2 * p + 3, 1, gsb)
            drain(gather_desc, 2 * p + 2, 0, gsa)
            extract(2 * p + 2, 0)
            fire(store_desc, 2 * p + 2, 0, ssa)
            return carry

        lax.fori_loop(0, ngroups // 2 - 1, pair_body, 0)

        drain(store_desc, ngroups - 2, 0, ssa)
        drain(gather_desc, ngroups - 1, 1, gsb)
        extract(ngroups - 1, 1)
        fire(store_desc, ngroups - 1, 1, ssb)
        drain(store_desc, ngroups - 1, 1, ssb)

    return k


def kernel(x, table):
    bx, f = x.shape
    n, d = table.shape
    nidx = bx * f
    x2 = x.reshape(nidx // 128, 128)
    t2 = table.reshape(n // 4, 128)
    out2 = _make_gather(nidx, n // 4)(t2, x2)
    return out2.reshape(bx, f, d)


# speedup vs baseline: 1.6002x; 1.6002x over previous
import functools

import jax
import jax.numpy as jnp
from jax import lax
from jax.experimental import pallas as pl
from jax.experimental.pallas import tpu as pltpu
from jax.experimental.pallas import tpu_sc as plsc

_NC = 2
_NS = 16
_NW = _NC * _NS
_CR = 16
_K = 2


@functools.cache
def _make_gather(bx: int, f: int, d: int):
    xr = bx // _NW
    nchunk = xr // _CR
    ngroups = nchunk // _K
    assert xr % _CR == 0 and nchunk % _K == 0 and ngroups % 2 == 0 and ngroups >= 4
    mesh = plsc.VectorSubcoreMesh(core_axis_name="c", subcore_axis_name="s")

    @functools.partial(
        pl.kernel,
        mesh=mesh,
        out_type=jax.ShapeDtypeStruct((bx, f, d), jnp.float32),
        scratch_types=[
            pltpu.VMEM((xr, f), jnp.int32),
            pltpu.VMEM((2, _K, _CR, f, d), jnp.float32),
            pltpu.SemaphoreType.DMA,
            pltpu.SemaphoreType.DMA,
            pltpu.SemaphoreType.DMA,
            pltpu.SemaphoreType.DMA,
        ],
        compiler_params=pltpu.CompilerParams(use_tc_tiling_on_sc=False),
    )
    def k(table_hbm, x_hbm, out_hbm, idx_v, rows_v, gsa, gsb, ssa, ssb):
        wid = lax.axis_index("s") * _NC + lax.axis_index("c")
        base = wid * xr
        pltpu.sync_copy(x_hbm.at[pl.ds(base, xr)], idx_v)

        def gather_descs(grp, bufset, b, sem):
            return [
                pltpu.make_async_copy(
                    table_hbm.at[idx_v.at[(grp * _K + b) * _CR + r]],
                    rows_v.at[bufset, b, r],
                    sem,
                )
                for r in range(_CR)
            ]

        def store_desc(grp, bufset, b, sem):
            return pltpu.make_async_copy(
                rows_v.at[bufset, b],
                out_hbm.at[pl.ds(base + (grp * _K + b) * _CR, _CR)],
                sem,
            )

        def _descs(desc_fn, grp, bufset, sem):
            out = []
            for b in range(_K):
                d_ = desc_fn(grp, bufset, b, sem)
                out.extend(d_ if isinstance(d_, list) else [d_])
            return out

        def fire(desc_fn, grp, bufset, sem):
            for d_ in _descs(desc_fn, grp, bufset, sem):
                d_.start()

        def drain(desc_fn, grp, bufset, sem):
            for d_ in _descs(desc_fn, grp, bufset, sem):
                d_.wait()

        fire(gather_descs, 0, 0, gsa)
        fire(gather_descs, 1, 1, gsb)
        drain(gather_descs, 0, 0, gsa)
        fire(store_desc, 0, 0, ssa)

        def pair_body(p, carry):
            drain(store_desc, 2 * p, 0, ssa)
            fire(gather_descs, 2 * p + 2, 0, gsa)
            drain(gather_descs, 2 * p + 1, 1, gsb)
            fire(store_desc, 2 * p + 1, 1, ssb)
            drain(store_desc, 2 * p + 1, 1, ssb)
            fire(gather_descs, 2 * p + 3, 1, gsb)
            drain(gather_descs, 2 * p + 2, 0, gsa)
            fire(store_desc, 2 * p + 2, 0, ssa)
            return carry

        lax.fori_loop(0, ngroups // 2 - 1, pair_body, 0)

        drain(store_desc, ngroups - 2, 0, ssa)
        drain(gather_descs, ngroups - 1, 1, gsb)
        fire(store_desc, ngroups - 1, 1, ssb)
        drain(store_desc, ngroups - 1, 1, ssb)

    return k


def kernel(x, table):
    bx, f = x.shape
    n, d = table.shape
    return _make_gather(bx, f, d)(table, x)
